# Initial kernel scaffold; baseline (speedup 1.0000x reference)
import jax
import jax.numpy as jnp
from jax.experimental import pallas as pl


def kernel(h, graph_ids):
    raise NotImplementedError("write your pallas kernel here")


# speedup vs baseline: 3.1877x; 3.1877x over previous
import functools

import jax
import jax.numpy as jnp
from jax import lax
from jax.experimental import pallas as pl
from jax.experimental.pallas import tpu as pltpu
from jax.experimental.pallas import tpu_sc as plsc

NUM_GRAPHS = 256
N_NODES = 100000
D_FEAT = 128

NC = 2
NS = 16
DC = D_FEAT // NC
B = 128

ROWS_PER_SUB = 6256
LAST_ROWS = N_NODES - (NS - 1) * ROWS_PER_SUB
FULL_BLOCKS = 48
TAIL_MAIN = ROWS_PER_SUB - FULL_BLOCKS * B
TAIL_LAST = LAST_ROWS - FULL_BLOCKS * B


@functools.partial(
    pl.kernel,
    mesh=plsc.VectorSubcoreMesh(core_axis_name="c", subcore_axis_name="s"),
    out_type=jax.ShapeDtypeStruct((NUM_GRAPHS, D_FEAT), jnp.float32),
    compiler_params=pltpu.CompilerParams(use_tc_tiling_on_sc=False),
    scratch_types=[
        pltpu.VMEM((B,), jnp.int32),
        pltpu.VMEM((TAIL_MAIN,), jnp.int32),
        pltpu.VMEM((TAIL_LAST,), jnp.int32),
        pltpu.VMEM((B, DC), jnp.float32),
        pltpu.VMEM((TAIL_MAIN, DC), jnp.float32),
        pltpu.VMEM((TAIL_LAST, DC), jnp.float32),
        pltpu.VMEM((NUM_GRAPHS, DC), jnp.float32),
        pltpu.VMEM_SHARED((NUM_GRAPHS, DC), jnp.float32),
    ],
)
def _segsum_sc(h_hbm, gid_hbm, out_hbm,
               idx_b, idx_tm, idx_tl,
               rows_b, rows_tm, rows_tl,
               zero_v, shared_acc):
    c = lax.axis_index("c")
    s = lax.axis_index("s")
    col0 = c * DC
    base = s * ROWS_PER_SUB

    @pl.when(s == 0)
    def _():
        zv = jnp.zeros((16,), jnp.float32)

        def zero_body(r, _):
            for j in range(DC // 16):
                zero_v[r, pl.ds(j * 16, 16)] = zv
            return 0

        lax.fori_loop(0, NUM_GRAPHS, zero_body, 0)
        pltpu.sync_copy(zero_v, shared_acc)

    plsc.subcore_barrier()

    def block_body(blk, _):
        start = base + blk * B
        pltpu.sync_copy(gid_hbm.at[pl.ds(start, B)], idx_b)
        pltpu.sync_copy(h_hbm.at[pl.ds(start, B), pl.ds(col0, DC)], rows_b)
        pltpu.sync_copy(rows_b, shared_acc.at[idx_b], add=True)
        return 0

    lax.fori_loop(0, FULL_BLOCKS, block_body, 0)

    tail_start = base + FULL_BLOCKS * B

    @pl.when(s < NS - 1)
    def _():
        pltpu.sync_copy(gid_hbm.at[pl.ds(tail_start, TAIL_MAIN)], idx_tm)
        pltpu.sync_copy(h_hbm.at[pl.ds(tail_start, TAIL_MAIN), pl.ds(col0, DC)],
                        rows_tm)
        pltpu.sync_copy(rows_tm, shared_acc.at[idx_tm], add=True)

    @pl.when(s == NS - 1)
    def _():
        pltpu.sync_copy(gid_hbm.at[pl.ds(tail_start, TAIL_LAST)], idx_tl)
        pltpu.sync_copy(h_hbm.at[pl.ds(tail_start, TAIL_LAST), pl.ds(col0, DC)],
                        rows_tl)
        pltpu.sync_copy(rows_tl, shared_acc.at[idx_tl], add=True)

    plsc.subcore_barrier()

    pltpu.sync_copy(shared_acc.at[pl.ds(s * 16, 16)],
                    out_hbm.at[pl.ds(s * 16, 16), pl.ds(col0, DC)])


def kernel(h, graph_ids):
    return _segsum_sc(h, graph_ids.astype(jnp.int32))


# speedup vs baseline: 4.7276x; 1.4831x over previous
import functools

import jax
import jax.numpy as jnp
from jax import lax
from jax.experimental import pallas as pl
from jax.experimental.pallas import tpu as pltpu
from jax.experimental.pallas import tpu_sc as plsc

NUM_GRAPHS = 256
N_NODES = 100000
D_FEAT = 128

NC = 2
NS = 16
DC = D_FEAT // NC
B = 128

ROWS_PER_SUB = 6256
LAST_ROWS = N_NODES - (NS - 1) * ROWS_PER_SUB
FULL_BLOCKS = 48
NPAIRS = FULL_BLOCKS // 2
TAIL_MAIN = ROWS_PER_SUB - FULL_BLOCKS * B
TAIL_LAST = LAST_ROWS - FULL_BLOCKS * B
GROWS = NUM_GRAPHS // NS


@functools.partial(
    pl.kernel,
    mesh=plsc.VectorSubcoreMesh(core_axis_name="c", subcore_axis_name="s"),
    out_type=jax.ShapeDtypeStruct((NUM_GRAPHS, D_FEAT), jnp.float32),
    compiler_params=pltpu.CompilerParams(use_tc_tiling_on_sc=False),
    scratch_types=[
        pltpu.VMEM((FULL_BLOCKS, B), jnp.int32),
        pltpu.VMEM((TAIL_MAIN,), jnp.int32),
        pltpu.VMEM((TAIL_LAST,), jnp.int32),
        pltpu.VMEM((B, DC), jnp.float32),
        pltpu.VMEM((B, DC), jnp.float32),
        pltpu.VMEM((TAIL_MAIN, DC), jnp.float32),
        pltpu.VMEM((TAIL_LAST, DC), jnp.float32),
        pltpu.VMEM((GROWS, DC), jnp.float32),
        pltpu.VMEM_SHARED((NUM_GRAPHS, DC), jnp.float32),
        pltpu.SemaphoreType.DMA,
        pltpu.SemaphoreType.DMA,
        pltpu.SemaphoreType.DMA,
    ],
)
def _segsum_sc(h_hbm, gid_hbm, out_hbm,
               idx_all, idx_tm, idx_tl,
               rows_a, rows_b, rows_tm, rows_tl,
               zero_v, shared_acc, sem_i, sem_a, sem_b):
    c = lax.axis_index("c")
    s = lax.axis_index("s")
    col0 = c * DC
    base = s * ROWS_PER_SUB

    def h_blk(start):
        return h_hbm.at[pl.ds(start, B), pl.ds(col0, DC)]

    def fire_idx(j, _):
        pltpu.async_copy(gid_hbm.at[pl.ds(base + j * B, B)], idx_all.at[j],
                         sem_i)
        return 0

    lax.fori_loop(0, FULL_BLOCKS, fire_idx, 0)
    pltpu.async_copy(h_blk(base), rows_a, sem_a)

    zv = jnp.zeros((16,), jnp.float32)

    def zero_body(r, _):
        for j in range(DC // 16):
            zero_v[r, pl.ds(j * 16, 16)] = zv
        return 0

    lax.fori_loop(0, GROWS, zero_body, 0)
    pltpu.sync_copy(zero_v, shared_acc.at[pl.ds(s * GROWS, GROWS)])

    def drain_idx(j, _):
        pltpu.make_async_copy(gid_hbm.at[pl.ds(base + j * B, B)],
                              idx_all.at[j], sem_i).wait()
        return 0

    lax.fori_loop(0, FULL_BLOCKS, drain_idx, 0)

    plsc.subcore_barrier()

    def pair_body(p, _):
        blk0 = 2 * p
        start0 = base + blk0 * B
        pltpu.make_async_copy(h_blk(base), rows_a, sem_a).wait()
        pltpu.async_copy(h_blk(start0 + B), rows_b, sem_b)
        pltpu.sync_copy(rows_a, shared_acc.at[idx_all.at[blk0]], add=True)
        pltpu.make_async_copy(h_blk(base), rows_b, sem_b).wait()

        @pl.when(p < NPAIRS - 1)
        def _():
            pltpu.async_copy(h_blk(start0 + 2 * B), rows_a, sem_a)

        pltpu.sync_copy(rows_b, shared_acc.at[idx_all.at[blk0 + 1]], add=True)
        return 0

    lax.fori_loop(0, NPAIRS, pair_body, 0)

    tail_start = base + FULL_BLOCKS * B

    @pl.when(s < NS - 1)
    def _():
        pltpu.sync_copy(gid_hbm.at[pl.ds(tail_start, TAIL_MAIN)], idx_tm)
        pltpu.sync_copy(h_hbm.at[pl.ds(tail_start, TAIL_MAIN), pl.ds(col0, DC)],
                        rows_tm)
        pltpu.sync_copy(rows_tm, shared_acc.at[idx_tm], add=True)

    @pl.when(s == NS - 1)
    def _():
        pltpu.sync_copy(gid_hbm.at[pl.ds(tail_start, TAIL_LAST)], idx_tl)
        pltpu.sync_copy(h_hbm.at[pl.ds(tail_start, TAIL_LAST), pl.ds(col0, DC)],
                        rows_tl)
        pltpu.sync_copy(rows_tl, shared_acc.at[idx_tl], add=True)

    plsc.subcore_barrier()

    pltpu.sync_copy(shared_acc.at[pl.ds(s * GROWS, GROWS)],
                    out_hbm.at[pl.ds(s * GROWS, GROWS), pl.ds(col0, DC)])


def kernel(h, graph_ids):
    return _segsum_sc(h, graph_ids.astype(jnp.int32))
